# baseline (speedup 1.0000x reference)
import functools

import jax
import jax.numpy as jnp
from jax import lax
from jax.experimental import pallas as pl
from jax.experimental.pallas import tpu as pltpu
from jax.experimental.pallas import tpu_sc as plsc

_NBUF = 8
_BB = 128


@functools.lru_cache(maxsize=None)
def _build_gather(batch: int, seq: int, vocab: int, dim: int):
    info = plsc.get_sparse_core_info()
    nc, ns, lanes = info.num_cores, info.num_subcores, info.num_lanes
    nw = nc * ns
    assert dim % lanes == 0 and batch % nw == 0
    b_per_w = batch // nw
    seq_p = -(-seq // 8) * 8
    nbuf = _NBUF
    assert b_per_w % nbuf == 0
    rounds = b_per_w

    mesh = plsc.VectorSubcoreMesh(core_axis_name="c", subcore_axis_name="s")

    @functools.partial(
        pl.kernel,
        mesh=mesh,
        out_type=jax.ShapeDtypeStruct((batch, seq_p, 128), jnp.float32),
        scratch_types=[
            pltpu.VMEM((b_per_w, 128), jnp.int32),
            pltpu.VMEM((nbuf, seq_p, 128), jnp.float32),
        ]
        + [pltpu.SemaphoreType.DMA] * (2 * _NBUF),
    )
    def gather_kernel(table_hbm, x_hbm, out_hbm, idx_v, bufs, *sems):
        sem_g = sems[:nbuf]
        sem_s = sems[nbuf:]
        wid = lax.axis_index("s") * nc + lax.axis_index("c")
        b0 = wid * b_per_w
        pltpu.sync_copy(x_hbm.at[pl.ds(b0, b_per_w)], idx_v)

        def gather_start(r, slot):
            pltpu.async_copy(
                table_hbm.at[idx_v.at[r, pl.ds(0, seq)]],
                bufs.at[slot, pl.ds(0, seq)],
                sem_g[slot],
            )

        def gather_wait(r, slot):
            pltpu.make_async_copy(
                table_hbm.at[idx_v.at[r, pl.ds(0, seq)]],
                bufs.at[slot, pl.ds(0, seq)],
                sem_g[slot],
            ).wait()

        def store_start(r, slot):
            return pltpu.async_copy(
                bufs.at[slot],
                out_hbm.at[b0 + r],
                sem_s[slot],
            )

        for slot in range(nbuf):
            gather_start(slot, slot)

        def step(g, carry):
            stores = []
            for slot in range(nbuf):
                gather_wait(g * nbuf + slot, slot)
                stores.append(store_start(g * nbuf + slot, slot))
            for slot in range(nbuf):
                stores[slot].wait()
                gather_start(g * nbuf + slot + nbuf, slot)
            return carry

        lax.fori_loop(0, rounds // nbuf - 1, step, 0)

        stores = []
        for slot in range(nbuf):
            r = rounds - nbuf + slot
            gather_wait(r, slot)
            stores.append(store_start(r, slot))
        for h in stores:
            h.wait()

    return gather_kernel


@functools.lru_cache(maxsize=None)
def _build_compact(batch: int, seq: int, seq_p: int, dim: int):
    bb = _BB
    assert batch % bb == 0

    def body(in_ref, out_ref):
        out_ref[...] = in_ref[:, :seq, :dim]

    return pl.pallas_call(
        body,
        grid=(batch // bb,),
        in_specs=[
            pl.BlockSpec((bb, seq_p, 128), lambda i: (i, 0, 0)),
        ],
        out_specs=pl.BlockSpec((bb, seq, dim), lambda i: (i, 0, 0)),
        out_shape=jax.ShapeDtypeStruct((batch, seq, dim), jnp.float32),
    )


def kernel(x, emb_wi):
    b, s = x.shape
    v, d = emb_wi.shape
    s_p = -(-s // 8) * 8
    run = _build_gather(b, s, v, d)
    compact = _build_compact(b, s, s_p, d)
    table_p = jnp.pad(emb_wi, ((0, 0), (0, 128 - d)))
    x_p = jnp.pad(x.astype(jnp.int32), ((0, 0), (0, 128 - s)))
    wide = run(table_p, x_p)
    return compact(wide)


# speedup vs baseline: 1.5721x; 1.5721x over previous
import functools

import jax
import jax.numpy as jnp
from jax import lax
from jax.experimental import pallas as pl
from jax.experimental.pallas import tpu as pltpu
from jax.experimental.pallas import tpu_sc as plsc

_NBUF = 8


@functools.lru_cache(maxsize=None)
def _build_gather(batch: int, seq: int, vocab: int, dim: int):
    info = plsc.get_sparse_core_info()
    nc, ns, lanes = info.num_cores, info.num_subcores, info.num_lanes
    nw = nc * ns
    assert dim % lanes == 0 and batch % nw == 0
    b_per_w = batch // nw
    seq_p = -(-seq // 8) * 8
    nbuf = _NBUF
    assert b_per_w % nbuf == 0
    rounds = b_per_w

    mesh = plsc.VectorSubcoreMesh(core_axis_name="c", subcore_axis_name="s")

    @functools.partial(
        pl.kernel,
        mesh=mesh,
        out_type=jax.ShapeDtypeStruct((batch, seq, 128), jnp.float32),
        scratch_types=[
            pltpu.VMEM((b_per_w, 128), jnp.int32),
            pltpu.VMEM((nbuf, seq, 128), jnp.float32),
        ]
        + [pltpu.SemaphoreType.DMA] * (2 * _NBUF),
    )
    def gather_kernel(table_hbm, x_hbm, out_hbm, idx_v, bufs, *sems):
        sem_g = sems[:nbuf]
        sem_s = sems[nbuf:]
        wid = lax.axis_index("s") * nc + lax.axis_index("c")
        b0 = wid * b_per_w
        pltpu.sync_copy(x_hbm.at[pl.ds(b0, b_per_w)], idx_v)

        def gather_start(r, slot):
            pltpu.async_copy(
                table_hbm.at[idx_v.at[r, pl.ds(0, seq)]],
                bufs.at[slot],
                sem_g[slot],
            )

        def gather_wait(r, slot):
            pltpu.make_async_copy(
                table_hbm.at[idx_v.at[r, pl.ds(0, seq)]],
                bufs.at[slot],
                sem_g[slot],
            ).wait()

        def store_start(r, slot):
            return pltpu.async_copy(
                bufs.at[slot],
                out_hbm.at[b0 + r],
                sem_s[slot],
            )

        for slot in range(nbuf):
            gather_start(slot, slot)

        def step(g, carry):
            stores = []
            for slot in range(nbuf):
                gather_wait(g * nbuf + slot, slot)
                stores.append(store_start(g * nbuf + slot, slot))
            for slot in range(nbuf):
                stores[slot].wait()
                gather_start(g * nbuf + slot + nbuf, slot)
            return carry

        lax.fori_loop(0, rounds // nbuf - 1, step, 0)

        stores = []
        for slot in range(nbuf):
            r = rounds - nbuf + slot
            gather_wait(r, slot)
            stores.append(store_start(r, slot))
        for h in stores:
            h.wait()

    return gather_kernel


def kernel(x, emb_wi):
    b, s = x.shape
    v, d = emb_wi.shape
    run = _build_gather(b, s, v, d)
    table_p = jnp.pad(emb_wi, ((0, 0), (0, 128 - d)))
    x_p = jnp.pad(x.astype(jnp.int32), ((0, 0), (0, 128 - s)))
    wide = run(table_p, x_p)
    return wide[:, :, :d]
